# Initial kernel scaffold; baseline (speedup 1.0000x reference)
import jax
import jax.numpy as jnp
from jax.experimental import pallas as pl


def kernel(x0, x1, edge_index0, edge_index1, edge_weight0, edge_weight1, adj0, adj1, enc0_W1, enc0_b1, enc0_W2, enc0_b2, enc1_W1, enc1_b1, enc1_W2, enc1_b2, attn_w, ref_W1, ref_b1, ref_W2, ref_b2, dec_W0, dec_W1, cls_W1, cls_b1, cls_W2, cls_b2):
    raise NotImplementedError("write your pallas kernel here")


# speedup vs baseline: 7.9532x; 7.9532x over previous
import functools

import jax
import jax.numpy as jnp
from jax import lax
from jax.experimental import pallas as pl
from jax.experimental.pallas import tpu as pltpu
from jax.experimental.pallas import tpu_sc as plsc

_N = 4096
_E = 65536
_HID = 256

_NC = 2
_NS = 16
_L = 16

_EPC = _E // _NC
_EPT = _EPC // _NS
_K = 128
_NCH = _EPT // _K
_RPT = _N // _NS

_mesh = plsc.VectorSubcoreMesh(
    core_axis_name="c", subcore_axis_name="s", num_cores=_NC, num_subcores=_NS
)


@functools.partial(
    pl.kernel,
    out_type=[
        jax.ShapeDtypeStruct((_NC, _N, _L), jnp.float32),
        jax.ShapeDtypeStruct((_NC, _N, _L), jnp.float32),
    ],
    mesh=_mesh,
    scratch_types=[
        pltpu.VMEM_SHARED((_N, _L), jnp.float32),
        pltpu.VMEM_SHARED((_N, _L), jnp.float32),
        pltpu.VMEM((_K,), jnp.int32),
        pltpu.VMEM((_K,), jnp.float32),
        pltpu.VMEM((_K, _L), jnp.float32),
        pltpu.VMEM((_K, _L), jnp.float32),
    ],
)
def _sc_deg(col0, ew0, col1, ew1, degp0, degp1, acc0, acc1, colv, ewv, buf,
            zbuf):
    c = lax.axis_index("c")
    s = lax.axis_index("s")

    zero = jnp.zeros((_L,), jnp.float32)
    for j in range(_K):
        zbuf[j] = zero
    for half in range(_RPT // _K):
        pltpu.sync_copy(zbuf, acc0.at[pl.ds(s * _RPT + half * _K, _K)])
        pltpu.sync_copy(zbuf, acc1.at[pl.ds(s * _RPT + half * _K, _K)])
    plsc.subcore_barrier()

    lane = lax.iota(jnp.int32, _L)
    for g, (colh, ewh, acc) in enumerate(((col0, ew0, acc0), (col1, ew1, acc1))):
        @pl.loop(0, _NCH)
        def _chunk(k):
            base = c * _EPC + s * _EPT + k * _K
            pltpu.sync_copy(colh.at[pl.ds(base, _K)], colv)
            pltpu.sync_copy(ewh.at[pl.ds(base, _K)], ewv)

            @pl.loop(0, _K // _L)
            def _fill(j16):
                wv = ewv[pl.ds(j16 * _L, _L)]
                for u in range(_L):
                    buf[j16 * _L + u] = jnp.where(lane < 8, wv[u],
                                                  jnp.float32(1.0))

            pltpu.sync_copy(buf, acc.at[colv], add=True)

    plsc.subcore_barrier()
    pltpu.sync_copy(acc0.at[pl.ds(s * _RPT, _RPT)], degp0.at[c, pl.ds(s * _RPT, _RPT)])
    pltpu.sync_copy(acc1.at[pl.ds(s * _RPT, _RPT)], degp1.at[c, pl.ds(s * _RPT, _RPT)])


def _make_agg(num_graphs, weighted):
    out_type = [
        jax.ShapeDtypeStruct((_NC * _N, _HID), jnp.float32)
        for _ in range(num_graphs)
    ]
    scratch = [
        pltpu.VMEM((_K,), jnp.int32),
        pltpu.VMEM((_K,), jnp.int32),
        pltpu.VMEM((_K,), jnp.float32),
        pltpu.VMEM((_K, _HID), jnp.float32),
        pltpu.SemaphoreType.DMA,
        pltpu.SemaphoreType.DMA,
    ]

    @functools.partial(pl.kernel, out_type=out_type, mesh=_mesh,
                       scratch_types=scratch)
    def _agg(*refs):
        nin = (4 if weighted else 3) * num_graphs
        ins = refs[:nin]
        outs = refs[nin:nin + num_graphs]
        rowv, colv, ewv, rows, gsem, ssem = refs[nin + num_graphs:]

        c = lax.axis_index("c")
        s = lax.axis_index("s")

        zero = jnp.zeros((_L,), jnp.float32)

        for g in range(num_graphs):
            if weighted:
                rowh, colh, ewh, xwsh = ins[4 * g:4 * g + 4]
            else:
                rowh, colh, xwsh = ins[3 * g:3 * g + 3]
                ewh = None
            out = outs[g]

            @pl.loop(0, _K)
            def _zero(j):
                for q in range(_HID // _L):
                    rows[j, pl.ds(q * _L, _L)] = zero

            for half in range(_RPT // _K):
                pltpu.sync_copy(
                    rows,
                    out.at[pl.ds(c * _N + s * _RPT + half * _K, _K)])
            plsc.subcore_barrier()

            @pl.loop(0, _NCH)
            def _chunk(k):
                base = c * _EPC + s * _EPT + k * _K
                pltpu.sync_copy(rowh.at[pl.ds(base, _K)], rowv)
                pltpu.sync_copy(colh.at[pl.ds(base, _K)], colv)
                if weighted:
                    pltpu.sync_copy(ewh.at[pl.ds(base, _K)], ewv)
                pltpu.async_copy(xwsh.at[rowv], rows, gsem).wait()

                if weighted:
                    @pl.loop(0, _K // _L)
                    def _scale(j16):
                        wv = ewv[pl.ds(j16 * _L, _L)]
                        for u in range(_L):
                            j = j16 * _L + u
                            for q in range(_HID // _L):
                                sl = pl.ds(q * _L, _L)
                                rows[j, sl] = rows[j, sl] * wv[u]

                @pl.loop(0, _K // _L)
                def _adj(j16):
                    sl = pl.ds(j16 * _L, _L)
                    colv[sl] = colv[sl] + c * _N

                pltpu.async_copy(rows, out.at[colv], ssem, add=True).wait()

            plsc.subcore_barrier()

    return _agg


_sc_agg2 = _make_agg(2, True)
_sc_agg1 = _make_agg(1, False)


_R = 512
_GRID = _N // _R

_f32 = jnp.float32


def _blk(shape, index_map):
    return pl.BlockSpec(shape, index_map)


def _full(shape):
    return pl.BlockSpec(shape, lambda i: tuple(0 for _ in shape))


def _rows(width=_HID):
    return pl.BlockSpec((_R, width), lambda i: (i, 0))


def _parts_spec():
    return pl.BlockSpec((_NC, _R, _HID), lambda i: (0, i, 0))


def _dot(a, b):
    return jnp.dot(a, b, preferred_element_type=_f32)


def _tc_prep_body(degp0, degp1, x0, x1, w01, w11,
                  dinv0, dinv1, dinvr, xws0, xws1):
    d0 = degp0[0, :, 0:1] + degp0[1, :, 0:1] + 1.0
    dr = degp0[0, :, 8:9] + degp0[1, :, 8:9] + 1.0
    d1 = degp1[0, :, 0:1] + degp1[1, :, 0:1] + 1.0
    i0 = lax.rsqrt(d0)
    i1 = lax.rsqrt(d1)
    ir = lax.rsqrt(dr)
    dinv0[...] = i0
    dinv1[...] = i1
    dinvr[...] = ir
    xws0[...] = i0 * _dot(x0[...], w01[...])
    xws1[...] = i1 * _dot(x1[...], w11[...])


def _tc_prep(degp0, degp1, x0, x1, w01, w11):
    return pl.pallas_call(
        _tc_prep_body,
        grid=(_GRID,),
        in_specs=[pl.BlockSpec((_NC, _R, _L), lambda i: (0, i, 0)),
                  pl.BlockSpec((_NC, _R, _L), lambda i: (0, i, 0)),
                  _rows(), _rows(),
                  _full((_HID, _HID)), _full((_HID, _HID))],
        out_specs=[pl.BlockSpec((_R, 1), lambda i: (i, 0)),
                   pl.BlockSpec((_R, 1), lambda i: (i, 0)),
                   pl.BlockSpec((_R, 1), lambda i: (i, 0)),
                   _rows(), _rows()],
        out_shape=[jax.ShapeDtypeStruct((_N, 1), _f32)] * 3
        + [jax.ShapeDtypeStruct((_N, _HID), _f32)] * 2,
    )(degp0, degp1, x0, x1, w01, w11)


def _tc_post1_body(p0, xws0, b0, dinv0, w02, p1, xws1, b1, dinv1, w12,
                   o0, o1):
    t0 = dinv0[...] * (p0[0] + p0[1] + xws0[...]) + b0[...]
    o0[...] = dinv0[...] * _dot(jax.nn.relu(t0), w02[...])
    t1 = dinv1[...] * (p1[0] + p1[1] + xws1[...]) + b1[...]
    o1[...] = dinv1[...] * _dot(jax.nn.relu(t1), w12[...])


def _tc_post1(p0, xws0, b0, dinv0, w02, p1, xws1, b1, dinv1, w12):
    return pl.pallas_call(
        _tc_post1_body,
        grid=(_GRID,),
        in_specs=[_parts_spec(), _rows(), _full((1, _HID)),
                  pl.BlockSpec((_R, 1), lambda i: (i, 0)),
                  _full((_HID, _HID)),
                  _parts_spec(), _rows(), _full((1, _HID)),
                  pl.BlockSpec((_R, 1), lambda i: (i, 0)),
                  _full((_HID, _HID))],
        out_specs=[_rows(), _rows()],
        out_shape=[jax.ShapeDtypeStruct((_N, _HID), _f32)] * 2,
    )(p0, xws0, b0, dinv0, w02, p1, xws1, b1, dinv1, w12)


def _tc_fuse_body(p0, xws0, b0, dinv0, p1, xws1, b1, dinv1, aw, wr1, dinvr,
                  h0o, h1o, xwsro):
    h0 = dinv0[...] * (p0[0] + p0[1] + xws0[...]) + b0[...]
    h1 = dinv1[...] * (p1[0] + p1[1] + xws1[...]) + b1[...]
    h0o[...] = h0
    h1o[...] = h1
    a = aw[...]
    m = jnp.max(a)
    e = jnp.exp(a - m)
    w = e / jnp.sum(e)
    fused = w[0, 0] * h0 + w[0, 1] * h1
    xwsro[...] = dinvr[...] * _dot(fused, wr1[...])


def _tc_fuse(p0, xws0, b0, dinv0, p1, xws1, b1, dinv1, aw, wr1, dinvr):
    return pl.pallas_call(
        _tc_fuse_body,
        grid=(_GRID,),
        in_specs=[_parts_spec(), _rows(), _full((1, _HID)),
                  pl.BlockSpec((_R, 1), lambda i: (i, 0)),
                  _parts_spec(), _rows(), _full((1, _HID)),
                  pl.BlockSpec((_R, 1), lambda i: (i, 0)),
                  _full((1, 2)), _full((_HID, _HID)),
                  pl.BlockSpec((_R, 1), lambda i: (i, 0))],
        out_specs=[_rows(), _rows(), _rows()],
        out_shape=[jax.ShapeDtypeStruct((_N, _HID), _f32)] * 3,
    )(p0, xws0, b0, dinv0, p1, xws1, b1, dinv1, aw, wr1, dinvr)


def _tc_postref_body(p, xws, b, dinvr, wr2, o):
    t = dinvr[...] * (p[0] + p[1] + xws[...]) + b[...]
    o[...] = dinvr[...] * _dot(jax.nn.relu(t), wr2[...])


def _tc_postref(p, xws, b, dinvr, wr2):
    return pl.pallas_call(
        _tc_postref_body,
        grid=(_GRID,),
        in_specs=[_parts_spec(), _rows(), _full((1, _HID)),
                  pl.BlockSpec((_R, 1), lambda i: (i, 0)),
                  _full((_HID, _HID))],
        out_specs=[_rows()],
        out_shape=[jax.ShapeDtypeStruct((_N, _HID), _f32)],
    )(p, xws, b, dinvr, wr2)[0]


def _tc_final_body(p, xws, b, dinvr, cw1, cb1, cw2, cb2, zo, lo):
    z = dinvr[...] * (p[0] + p[1] + xws[...]) + b[...]
    zo[...] = z
    t = jax.nn.relu(_dot(z, cw1[...]) + cb1[...])
    lo[...] = _dot(t, cw2[...]) + cb2[...]


def _tc_final(p, xws, b, dinvr, cw1, cb1, cw2, cb2):
    nc = cw2.shape[1]
    return pl.pallas_call(
        _tc_final_body,
        grid=(_GRID,),
        in_specs=[_parts_spec(), _rows(), _full((1, _HID)),
                  pl.BlockSpec((_R, 1), lambda i: (i, 0)),
                  _full((_HID, _HID // 2)), _full((1, _HID // 2)),
                  _full((_HID // 2, nc)), _full((1, nc))],
        out_specs=[_rows(), pl.BlockSpec((_R, nc), lambda i: (i, 0))],
        out_shape=[jax.ShapeDtypeStruct((_N, _HID), _f32),
                   jax.ShapeDtypeStruct((_N, nc), _f32)],
    )(p, xws, b, dinvr, cw1, cb1, cw2, cb2)


_RB = 256


def _tc_decode_body(zb, zfull, w0, w1, r0, r1):
    zw0 = _dot(zb[...], w0[...])
    zw1 = _dot(zb[...], w1[...])
    zf = zfull[...]
    dims = (((1,), (1,)), ((), ()))
    s0 = lax.dot_general(zw0, zf, dims, preferred_element_type=_f32)
    s1 = lax.dot_general(zw1, zf, dims, preferred_element_type=_f32)
    r0[...] = jax.nn.sigmoid(s0)
    r1[...] = jax.nn.sigmoid(s1)


def _tc_decode(z, w0, w1):
    return pl.pallas_call(
        _tc_decode_body,
        grid=(_N // _RB,),
        in_specs=[pl.BlockSpec((_RB, _HID), lambda i: (i, 0)),
                  _full((_N, _HID)),
                  _full((_HID, _HID)), _full((_HID, _HID))],
        out_specs=[pl.BlockSpec((_RB, _N), lambda i: (i, 0)),
                   pl.BlockSpec((_RB, _N), lambda i: (i, 0))],
        out_shape=[jax.ShapeDtypeStruct((_N, _N), _f32)] * 2,
    )(z, z, w0, w1)


def kernel(x0, x1, edge_index0, edge_index1, edge_weight0, edge_weight1,
           adj0, adj1,
           enc0_W1, enc0_b1, enc0_W2, enc0_b2,
           enc1_W1, enc1_b1, enc1_W2, enc1_b2,
           attn_w, ref_W1, ref_b1, ref_W2, ref_b2,
           dec_W0, dec_W1, cls_W1, cls_b1, cls_W2, cls_b2):
    row0 = edge_index0[0]
    col0 = edge_index0[1]
    row1 = edge_index1[0]
    col1 = edge_index1[1]

    b2d = lambda b: b.reshape(1, -1)
    p3d = lambda p: p.reshape(_NC, _N, _HID)

    degp0, degp1 = _sc_deg(col0, edge_weight0, col1, edge_weight1)
    dinv0, dinv1, dinvr, xws0, xws1 = _tc_prep(
        degp0, degp1, x0, x1, enc0_W1, enc1_W1)

    p0, p1 = _sc_agg2(row0, col0, edge_weight0, xws0,
                      row1, col1, edge_weight1, xws1)
    xws0b, xws1b = _tc_post1(p3d(p0), xws0, b2d(enc0_b1), dinv0, enc0_W2,
                             p3d(p1), xws1, b2d(enc1_b1), dinv1, enc1_W2)

    q0, q1 = _sc_agg2(row0, col0, edge_weight0, xws0b,
                      row1, col1, edge_weight1, xws1b)
    h0, h1, xwsr = _tc_fuse(p3d(q0), xws0b, b2d(enc0_b2), dinv0,
                            p3d(q1), xws1b, b2d(enc1_b2), dinv1,
                            attn_w.reshape(1, 2), ref_W1, dinvr)

    (pr,) = _sc_agg1(row0, col0, xwsr)
    xwsr2 = _tc_postref(p3d(pr), xwsr, b2d(ref_b1), dinvr, ref_W2)

    (qr,) = _sc_agg1(row0, col0, xwsr2)
    z, logits = _tc_final(p3d(qr), xwsr2, b2d(ref_b2), dinvr,
                          cls_W1, b2d(cls_b1), cls_W2, b2d(cls_b2))

    r0, r1 = _tc_decode(z, dec_W0, dec_W1)
    return (logits, (r0, r1), (h0, h1))


# speedup vs baseline: 10.6934x; 1.3445x over previous
import functools

import jax
import jax.numpy as jnp
from jax import lax
from jax.experimental import pallas as pl
from jax.experimental.pallas import tpu as pltpu
from jax.experimental.pallas import tpu_sc as plsc

_N = 4096
_E = 65536
_HID = 256

_NC = 2
_NS = 16
_L = 16

_EPC = _E // _NC
_EPT = _EPC // _NS
_K = 128
_NCH = _EPT // _K
_RPT = _N // _NS

_mesh = plsc.VectorSubcoreMesh(
    core_axis_name="c", subcore_axis_name="s", num_cores=_NC, num_subcores=_NS
)


@functools.partial(
    pl.kernel,
    out_type=[
        jax.ShapeDtypeStruct((_NC, _N, _L), jnp.float32),
        jax.ShapeDtypeStruct((_NC, _N, _L), jnp.float32),
    ],
    mesh=_mesh,
    scratch_types=[
        pltpu.VMEM_SHARED((_N, _L), jnp.float32),
        pltpu.VMEM_SHARED((_N, _L), jnp.float32),
        pltpu.VMEM((_K,), jnp.int32),
        pltpu.VMEM((_K,), jnp.float32),
        pltpu.VMEM((_K, _L), jnp.float32),
        pltpu.VMEM((_K, _L), jnp.float32),
    ],
)
def _sc_deg(col0, ew0, col1, ew1, degp0, degp1, acc0, acc1, colv, ewv, buf,
            zbuf):
    c = lax.axis_index("c")
    s = lax.axis_index("s")

    zero = jnp.zeros((_L,), jnp.float32)
    for j in range(_K):
        zbuf[j] = zero
    for half in range(_RPT // _K):
        pltpu.sync_copy(zbuf, acc0.at[pl.ds(s * _RPT + half * _K, _K)])
        pltpu.sync_copy(zbuf, acc1.at[pl.ds(s * _RPT + half * _K, _K)])
    plsc.subcore_barrier()

    lane = lax.iota(jnp.int32, _L)
    for g, (colh, ewh, acc) in enumerate(((col0, ew0, acc0), (col1, ew1, acc1))):
        @pl.loop(0, _NCH)
        def _chunk(k):
            base = c * _EPC + s * _EPT + k * _K
            pltpu.sync_copy(colh.at[pl.ds(base, _K)], colv)
            pltpu.sync_copy(ewh.at[pl.ds(base, _K)], ewv)

            @pl.loop(0, _K // _L)
            def _fill(j16):
                wv = ewv[pl.ds(j16 * _L, _L)]
                for u in range(_L):
                    buf[j16 * _L + u] = jnp.where(lane < 8, wv[u],
                                                  jnp.float32(1.0))

            pltpu.sync_copy(buf, acc.at[colv], add=True)

    plsc.subcore_barrier()
    pltpu.sync_copy(acc0.at[pl.ds(s * _RPT, _RPT)], degp0.at[c, pl.ds(s * _RPT, _RPT)])
    pltpu.sync_copy(acc1.at[pl.ds(s * _RPT, _RPT)], degp1.at[c, pl.ds(s * _RPT, _RPT)])


def _make_agg(num_graphs, weighted):
    out_type = [
        jax.ShapeDtypeStruct((_NC * _N, _HID), jnp.float32)
        for _ in range(num_graphs)
    ]
    scratch = [
        pltpu.VMEM((_EPT,), jnp.int32),
        pltpu.VMEM((_EPT,), jnp.int32),
        pltpu.VMEM((_EPT,), jnp.float32),
        pltpu.VMEM((_K, _HID), jnp.float32),
        pltpu.VMEM((_K, _HID), jnp.float32),
        pltpu.VMEM((_K,), jnp.int32),
        pltpu.VMEM((_K,), jnp.int32),
        pltpu.VMEM((_K,), jnp.int32),
        pltpu.SemaphoreType.DMA,
        pltpu.SemaphoreType.DMA,
    ]

    @functools.partial(pl.kernel, out_type=out_type, mesh=_mesh,
                       scratch_types=scratch)
    def _agg(*refs):
        nin = (4 if weighted else 3) * num_graphs
        ins = refs[:nin]
        outs = refs[nin:nin + num_graphs]
        (rowv, colv, ewv, rows_a, rows_b, rowcur_a, rowcur_b, colcur,
         gsa, gsb) = refs[nin + num_graphs:]
        bufs = (rows_a, rows_b)
        rowcurs = (rowcur_a, rowcur_b)
        sems = (gsa, gsb)

        def _vcopy_row(src1d, kk, dst1d):
            for q in range(_K // _L):
                dst1d[pl.ds(q * _L, _L)] = src1d[pl.ds(kk * _K + q * _L, _L)]

        c = lax.axis_index("c")
        s = lax.axis_index("s")
        tb = c * _EPC + s * _EPT

        zero = jnp.zeros((_L,), jnp.float32)

        for g in range(num_graphs):
            if weighted:
                rowh, colh, ewh, xwsh = ins[4 * g:4 * g + 4]
            else:
                rowh, colh, xwsh = ins[3 * g:3 * g + 3]
                ewh = None
            out = outs[g]

            @pl.loop(0, _K)
            def _zero(j):
                for q in range(_HID // _L):
                    rows_a[j, pl.ds(q * _L, _L)] = zero

            for half in range(_RPT // _K):
                pltpu.sync_copy(
                    rows_a,
                    out.at[pl.ds(c * _N + s * _RPT + half * _K, _K)])

            pltpu.sync_copy(rowh.at[pl.ds(tb, _EPT)], rowv)
            pltpu.sync_copy(colh.at[pl.ds(tb, _EPT)], colv)
            if weighted:
                pltpu.sync_copy(ewh.at[pl.ds(tb, _EPT)], ewv)
            cn = c * _N

            @pl.loop(0, _EPT // _L)
            def _adj(k):
                sl = pl.ds(k * _L, _L)
                colv[sl] = colv[sl] + cn

            plsc.subcore_barrier()

            _vcopy_row(rowv, 0, rowcur_a)
            pltpu.async_copy(xwsh.at[rowcur_a], rows_a, gsa)
            _vcopy_row(rowv, 1, rowcur_b)
            pltpu.async_copy(xwsh.at[rowcur_b], rows_b, gsb)

            @pl.loop(0, _NCH, step=2)
            def _chunk(k):
                for b in range(2):
                    kk = k + b
                    buf = bufs[b]
                    sem = sems[b]
                    rowcur = rowcurs[b]
                    pltpu.make_async_copy(xwsh.at[rowcur], buf, sem).wait()

                    if weighted:
                        @pl.loop(0, _K // _L)
                        def _scale(j16):
                            wv = ewv[pl.ds(kk * _K + j16 * _L, _L)]
                            for u in range(_L):
                                j = j16 * _L + u
                                for q in range(_HID // _L):
                                    sl = pl.ds(q * _L, _L)
                                    buf[j, sl] = buf[j, sl] * wv[u]

                    _vcopy_row(colv, kk, colcur)
                    pltpu.sync_copy(buf, out.at[colcur], add=True)
                    kk2 = jnp.minimum(kk + 2, _NCH - 1)
                    _vcopy_row(rowv, kk2, rowcur)
                    pltpu.async_copy(xwsh.at[rowcur], buf, sem)

            pltpu.make_async_copy(xwsh.at[rowcur_a], rows_a, gsa).wait()
            pltpu.make_async_copy(xwsh.at[rowcur_b], rows_b, gsb).wait()

    return _agg


_sc_agg2 = _make_agg(2, True)
_sc_agg1 = _make_agg(1, False)


_R = 512
_GRID = _N // _R

_f32 = jnp.float32


def _blk(shape, index_map):
    return pl.BlockSpec(shape, index_map)


def _full(shape):
    return pl.BlockSpec(shape, lambda i: tuple(0 for _ in shape))


def _rows(width=_HID):
    return pl.BlockSpec((_R, width), lambda i: (i, 0))


def _parts_spec():
    return pl.BlockSpec((_NC, _R, _HID), lambda i: (0, i, 0))


def _dot(a, b):
    return jnp.dot(a, b, preferred_element_type=_f32)


def _tc_prep_body(degp0, degp1, x0, x1, w01, w11,
                  dinv0, dinv1, dinvr, xws0, xws1):
    d0 = degp0[0, :, 0:1] + degp0[1, :, 0:1] + 1.0
    dr = degp0[0, :, 8:9] + degp0[1, :, 8:9] + 1.0
    d1 = degp1[0, :, 0:1] + degp1[1, :, 0:1] + 1.0
    i0 = lax.rsqrt(d0)
    i1 = lax.rsqrt(d1)
    ir = lax.rsqrt(dr)
    dinv0[...] = i0
    dinv1[...] = i1
    dinvr[...] = ir
    xws0[...] = i0 * _dot(x0[...], w01[...])
    xws1[...] = i1 * _dot(x1[...], w11[...])


def _tc_prep(degp0, degp1, x0, x1, w01, w11):
    return pl.pallas_call(
        _tc_prep_body,
        grid=(_GRID,),
        in_specs=[pl.BlockSpec((_NC, _R, _L), lambda i: (0, i, 0)),
                  pl.BlockSpec((_NC, _R, _L), lambda i: (0, i, 0)),
                  _rows(), _rows(),
                  _full((_HID, _HID)), _full((_HID, _HID))],
        out_specs=[pl.BlockSpec((_R, 1), lambda i: (i, 0)),
                   pl.BlockSpec((_R, 1), lambda i: (i, 0)),
                   pl.BlockSpec((_R, 1), lambda i: (i, 0)),
                   _rows(), _rows()],
        out_shape=[jax.ShapeDtypeStruct((_N, 1), _f32)] * 3
        + [jax.ShapeDtypeStruct((_N, _HID), _f32)] * 2,
    )(degp0, degp1, x0, x1, w01, w11)


def _tc_post1_body(p0, xws0, b0, dinv0, w02, p1, xws1, b1, dinv1, w12,
                   o0, o1):
    t0 = dinv0[...] * (p0[0] + p0[1] + xws0[...]) + b0[...]
    o0[...] = dinv0[...] * _dot(jax.nn.relu(t0), w02[...])
    t1 = dinv1[...] * (p1[0] + p1[1] + xws1[...]) + b1[...]
    o1[...] = dinv1[...] * _dot(jax.nn.relu(t1), w12[...])


def _tc_post1(p0, xws0, b0, dinv0, w02, p1, xws1, b1, dinv1, w12):
    return pl.pallas_call(
        _tc_post1_body,
        grid=(_GRID,),
        in_specs=[_parts_spec(), _rows(), _full((1, _HID)),
                  pl.BlockSpec((_R, 1), lambda i: (i, 0)),
                  _full((_HID, _HID)),
                  _parts_spec(), _rows(), _full((1, _HID)),
                  pl.BlockSpec((_R, 1), lambda i: (i, 0)),
                  _full((_HID, _HID))],
        out_specs=[_rows(), _rows()],
        out_shape=[jax.ShapeDtypeStruct((_N, _HID), _f32)] * 2,
    )(p0, xws0, b0, dinv0, w02, p1, xws1, b1, dinv1, w12)


def _tc_fuse_body(p0, xws0, b0, dinv0, p1, xws1, b1, dinv1, aw, wr1, dinvr,
                  h0o, h1o, xwsro):
    h0 = dinv0[...] * (p0[0] + p0[1] + xws0[...]) + b0[...]
    h1 = dinv1[...] * (p1[0] + p1[1] + xws1[...]) + b1[...]
    h0o[...] = h0
    h1o[...] = h1
    a = aw[...]
    m = jnp.max(a)
    e = jnp.exp(a - m)
    w = e / jnp.sum(e)
    fused = w[0, 0] * h0 + w[0, 1] * h1
    xwsro[...] = dinvr[...] * _dot(fused, wr1[...])


def _tc_fuse(p0, xws0, b0, dinv0, p1, xws1, b1, dinv1, aw, wr1, dinvr):
    return pl.pallas_call(
        _tc_fuse_body,
        grid=(_GRID,),
        in_specs=[_parts_spec(), _rows(), _full((1, _HID)),
                  pl.BlockSpec((_R, 1), lambda i: (i, 0)),
                  _parts_spec(), _rows(), _full((1, _HID)),
                  pl.BlockSpec((_R, 1), lambda i: (i, 0)),
                  _full((1, 2)), _full((_HID, _HID)),
                  pl.BlockSpec((_R, 1), lambda i: (i, 0))],
        out_specs=[_rows(), _rows(), _rows()],
        out_shape=[jax.ShapeDtypeStruct((_N, _HID), _f32)] * 3,
    )(p0, xws0, b0, dinv0, p1, xws1, b1, dinv1, aw, wr1, dinvr)


def _tc_postref_body(p, xws, b, dinvr, wr2, o):
    t = dinvr[...] * (p[0] + p[1] + xws[...]) + b[...]
    o[...] = dinvr[...] * _dot(jax.nn.relu(t), wr2[...])


def _tc_postref(p, xws, b, dinvr, wr2):
    return pl.pallas_call(
        _tc_postref_body,
        grid=(_GRID,),
        in_specs=[_parts_spec(), _rows(), _full((1, _HID)),
                  pl.BlockSpec((_R, 1), lambda i: (i, 0)),
                  _full((_HID, _HID))],
        out_specs=[_rows()],
        out_shape=[jax.ShapeDtypeStruct((_N, _HID), _f32)],
    )(p, xws, b, dinvr, wr2)[0]


def _tc_final_body(p, xws, b, dinvr, cw1, cb1, cw2, cb2, zo, lo):
    z = dinvr[...] * (p[0] + p[1] + xws[...]) + b[...]
    zo[...] = z
    t = jax.nn.relu(_dot(z, cw1[...]) + cb1[...])
    lo[...] = _dot(t, cw2[...]) + cb2[...]


def _tc_final(p, xws, b, dinvr, cw1, cb1, cw2, cb2):
    nc = cw2.shape[1]
    return pl.pallas_call(
        _tc_final_body,
        grid=(_GRID,),
        in_specs=[_parts_spec(), _rows(), _full((1, _HID)),
                  pl.BlockSpec((_R, 1), lambda i: (i, 0)),
                  _full((_HID, _HID // 2)), _full((1, _HID // 2)),
                  _full((_HID // 2, nc)), _full((1, nc))],
        out_specs=[_rows(), pl.BlockSpec((_R, nc), lambda i: (i, 0))],
        out_shape=[jax.ShapeDtypeStruct((_N, _HID), _f32),
                   jax.ShapeDtypeStruct((_N, nc), _f32)],
    )(p, xws, b, dinvr, cw1, cb1, cw2, cb2)


_RB = 256


def _tc_decode_body(zb, zfull, w0, w1, r0, r1):
    zw0 = _dot(zb[...], w0[...])
    zw1 = _dot(zb[...], w1[...])
    zf = zfull[...]
    dims = (((1,), (1,)), ((), ()))
    s0 = lax.dot_general(zw0, zf, dims, preferred_element_type=_f32)
    s1 = lax.dot_general(zw1, zf, dims, preferred_element_type=_f32)
    r0[...] = jax.nn.sigmoid(s0)
    r1[...] = jax.nn.sigmoid(s1)


def _tc_decode(z, w0, w1):
    return pl.pallas_call(
        _tc_decode_body,
        grid=(_N // _RB,),
        in_specs=[pl.BlockSpec((_RB, _HID), lambda i: (i, 0)),
                  _full((_N, _HID)),
                  _full((_HID, _HID)), _full((_HID, _HID))],
        out_specs=[pl.BlockSpec((_RB, _N), lambda i: (i, 0)),
                   pl.BlockSpec((_RB, _N), lambda i: (i, 0))],
        out_shape=[jax.ShapeDtypeStruct((_N, _N), _f32)] * 2,
    )(z, z, w0, w1)


def kernel(x0, x1, edge_index0, edge_index1, edge_weight0, edge_weight1,
           adj0, adj1,
           enc0_W1, enc0_b1, enc0_W2, enc0_b2,
           enc1_W1, enc1_b1, enc1_W2, enc1_b2,
           attn_w, ref_W1, ref_b1, ref_W2, ref_b2,
           dec_W0, dec_W1, cls_W1, cls_b1, cls_W2, cls_b2):
    row0 = edge_index0[0]
    col0 = edge_index0[1]
    row1 = edge_index1[0]
    col1 = edge_index1[1]
    ew0 = edge_weight0
    ew1 = edge_weight1

    b2d = lambda b: b.reshape(1, -1)
    p3d = lambda p: p.reshape(_NC, _N, _HID)

    degp0, degp1 = _sc_deg(col0, ew0, col1, ew1)
    dinv0, dinv1, dinvr, xws0, xws1 = _tc_prep(
        degp0, degp1, x0, x1, enc0_W1, enc1_W1)

    p0, p1 = _sc_agg2(row0, col0, ew0, xws0,
                      row1, col1, ew1, xws1)
    xws0b, xws1b = _tc_post1(p3d(p0), xws0, b2d(enc0_b1), dinv0, enc0_W2,
                             p3d(p1), xws1, b2d(enc1_b1), dinv1, enc1_W2)

    q0, q1 = _sc_agg2(row0, col0, ew0, xws0b,
                      row1, col1, ew1, xws1b)
    h0, h1, xwsr = _tc_fuse(p3d(q0), xws0b, b2d(enc0_b2), dinv0,
                            p3d(q1), xws1b, b2d(enc1_b2), dinv1,
                            attn_w.reshape(1, 2), ref_W1, dinvr)

    (pr,) = _sc_agg1(row0, col0, xwsr)
    xwsr2 = _tc_postref(p3d(pr), xwsr, b2d(ref_b1), dinvr, ref_W2)

    (qr,) = _sc_agg1(row0, col0, xwsr2)
    z, logits = _tc_final(p3d(qr), xwsr2, b2d(ref_b2), dinvr,
                          cls_W1, b2d(cls_b1), cls_W2, b2d(cls_b2))

    r0, r1 = _tc_decode(z, dec_W0, dec_W1)
    return (logits, (r0, r1), (h0, h1))
